# Initial kernel scaffold; baseline (speedup 1.0000x reference)
import jax
import jax.numpy as jnp
from jax.experimental import pallas as pl


def kernel(h, edge_index, W, b):
    raise NotImplementedError("write your pallas kernel here")


# speedup vs baseline: 1.8499x; 1.8499x over previous
import functools

import jax
import jax.numpy as jnp
from jax import lax
from jax.experimental import pallas as pl
from jax.experimental.pallas import tpu as pltpu
from jax.experimental.pallas import tpu_sc as plsc

N = 10000
E = 160000
D = 256

NC = 2
NS = 16
LANES = 16
NCHUNK = 8
CW = D // NCHUNK
EPW = E // NS
BATCH = 80
NB = EPW // BATCH
NP = 10240
ROWS_PER_SUB = NP // NS
FLUSH = 128
NFLUSH = ROWS_PER_SUB // FLUSH


def _project_kernel(h_ref, w_ref, b_ref, a_ref, bb_ref):
    p = jnp.dot(h_ref[...], w_ref[...], preferred_element_type=jnp.float32)
    a_ref[...] = p[:, :D]
    bb_ref[...] = p[:, D:] + b_ref[...]


def _node_projections(h, wab, b2):
    blk = 2000
    grid = N // blk
    return pl.pallas_call(
        _project_kernel,
        grid=(grid,),
        in_specs=[
            pl.BlockSpec((blk, D), lambda i: (i, 0)),
            pl.BlockSpec((D, 2 * D), lambda i: (0, 0)),
            pl.BlockSpec((1, D), lambda i: (0, 0)),
        ],
        out_specs=[
            pl.BlockSpec((blk, D), lambda i: (i, 0)),
            pl.BlockSpec((blk, D), lambda i: (i, 0)),
        ],
        out_shape=[
            jax.ShapeDtypeStruct((N, D), jnp.float32),
            jax.ShapeDtypeStruct((N, D), jnp.float32),
        ],
    )(h, wab, b2)


def _edge_body(atbl, htbl, btbl, eidx, out,
               sidx, didx, sraw, draw,
               abuf, hbuf, bbuf, obuf, zbuf, fbuf, wbuf,
               acc, sem_a, sem_h, sem_b):
    cid = lax.axis_index("c")
    sid = lax.axis_index("s")

    pltpu.sync_copy(eidx.at[0, sid], sraw)
    pltpu.sync_copy(eidx.at[1, sid], draw)

    zero16 = jnp.zeros((LANES,), jnp.float32)

    def zrow(r, carry):
        for j in range(2 * CW // LANES):
            zbuf[r, pl.ds(LANES * j, LANES)] = zero16
        return carry

    lax.fori_loop(0, FLUSH, zrow, 0)

    for kc in range(NCHUNK // NC):
        chunk = cid * (NCHUNK // NC) + kc

        def adj(r, carry):
            for j in range(BATCH // LANES):
                sl = pl.ds(LANES * j, LANES)
                sidx[r, sl] = sraw[r, sl] * NCHUNK + chunk
                didx[r, sl] = draw[r, sl] * NCHUNK + chunk
            return carry

        lax.fori_loop(0, NB, adj, 0)

        for z in range(NFLUSH):
            pltpu.sync_copy(zbuf, acc.at[pl.ds(sid * ROWS_PER_SUB + z * FLUSH, FLUSH)])
        plsc.subcore_barrier()

        def batch(bi, carry):
            ca = pltpu.async_copy(atbl.at[sidx.at[bi]], abuf, sem_a)
            ch = pltpu.async_copy(htbl.at[sidx.at[bi]], hbuf, sem_h)
            cb = pltpu.async_copy(btbl.at[didx.at[bi]], bbuf, sem_b)
            ca.wait()
            ch.wait()
            cb.wait()

            def row(r, c2):
                for j in range(CW // LANES):
                    sl = pl.ds(LANES * j, LANES)
                    x = abuf[r, sl] + bbuf[r, sl]
                    w = jnp.exp(jnp.maximum(x, x * 0.01))
                    obuf[r, sl] = w
                    obuf[r, pl.ds(CW + LANES * j, LANES)] = hbuf[r, sl] * w
                return c2

            lax.fori_loop(0, BATCH, row, 0)
            pltpu.sync_copy(obuf, acc.at[draw.at[bi]], add=True)
            return carry

        lax.fori_loop(0, NB, batch, 0)
        plsc.subcore_barrier()

        for z in range(NFLUSH):
            row0 = sid * ROWS_PER_SUB + z * FLUSH
            pltpu.sync_copy(acc.at[pl.ds(row0, FLUSH)], fbuf)

            def drow(r, carry):
                for j in range(CW // LANES):
                    sl = pl.ds(LANES * j, LANES)
                    den = fbuf[r, sl]
                    num = fbuf[r, pl.ds(CW + LANES * j, LANES)]
                    wbuf[r, sl] = jnp.where(den > 0.0, num / den, 0.0)
                return carry

            lax.fori_loop(0, FLUSH, drow, 0)
            pltpu.sync_copy(wbuf, out.at[chunk, pl.ds(row0, FLUSH)])


_edge_kernel = functools.partial(
    pl.kernel,
    out_type=jax.ShapeDtypeStruct((NCHUNK, NP, CW), jnp.float32),
    mesh=plsc.VectorSubcoreMesh(
        core_axis_name="c", subcore_axis_name="s", num_cores=NC, num_subcores=NS
    ),
    scratch_types=[
        pltpu.VMEM((NB, BATCH), jnp.int32),
        pltpu.VMEM((NB, BATCH), jnp.int32),
        pltpu.VMEM((NB, BATCH), jnp.int32),
        pltpu.VMEM((NB, BATCH), jnp.int32),
        pltpu.VMEM((BATCH, CW), jnp.float32),
        pltpu.VMEM((BATCH, CW), jnp.float32),
        pltpu.VMEM((BATCH, CW), jnp.float32),
        pltpu.VMEM((BATCH, 2 * CW), jnp.float32),
        pltpu.VMEM((FLUSH, 2 * CW), jnp.float32),
        pltpu.VMEM((FLUSH, 2 * CW), jnp.float32),
        pltpu.VMEM((FLUSH, CW), jnp.float32),
        pltpu.VMEM_SHARED((NP, 2 * CW), jnp.float32),
        pltpu.SemaphoreType.DMA,
        pltpu.SemaphoreType.DMA,
        pltpu.SemaphoreType.DMA,
    ],
    compiler_params=pltpu.CompilerParams(use_tc_tiling_on_sc=False),
)(_edge_body)


def kernel(h, edge_index, W, b):
    wab = jnp.concatenate([W[:, :D].T, W[:, D:].T], axis=1)
    b2 = b.reshape(1, D)
    a, bb = _node_projections(h, wab, b2)
    atbl = a.reshape(NCHUNK * N, CW)
    htbl = h.reshape(NCHUNK * N, CW)
    btbl = bb.reshape(NCHUNK * N, CW)
    eidx = edge_index.reshape(2, NS, NB, BATCH)
    outc = _edge_kernel(atbl, htbl, btbl, eidx)
    return outc[:, :N].transpose(1, 0, 2).reshape(N, D)


# speedup vs baseline: 2.9475x; 1.5933x over previous
import functools

import jax
import jax.numpy as jnp
from jax import lax
from jax.experimental import pallas as pl
from jax.experimental.pallas import tpu as pltpu
from jax.experimental.pallas import tpu_sc as plsc

N = 10000
E = 160000
D = 256

NC = 2
NS = 16
LANES = 16
NCHUNK = 8
CW = D // NCHUNK
SW = 2 * CW
EPW = E // NS
BATCH = 80
NB = EPW // BATCH
NP = 10240
ROWS_PER_SUB = NP // NS
FLUSH = 128
NFLUSH = ROWS_PER_SUB // FLUSH


def _project_kernel(h_ref, w_ref, b_ref, s_ref, bb_ref):
    hb = h_ref[...]
    p = jnp.dot(hb, w_ref[...], preferred_element_type=jnp.float32)
    parts = []
    for c in range(NCHUNK):
        parts.append(p[:, CW * c:CW * (c + 1)])
        parts.append(hb[:, CW * c:CW * (c + 1)])
    s_ref[...] = jnp.concatenate(parts, axis=1)
    bb_ref[...] = p[:, D:] + b_ref[...]


def _node_projections(h, wab, b2):
    blk = 2000
    grid = N // blk
    return pl.pallas_call(
        _project_kernel,
        grid=(grid,),
        in_specs=[
            pl.BlockSpec((blk, D), lambda i: (i, 0)),
            pl.BlockSpec((D, 2 * D), lambda i: (0, 0)),
            pl.BlockSpec((1, D), lambda i: (0, 0)),
        ],
        out_specs=[
            pl.BlockSpec((blk, 2 * D), lambda i: (i, 0)),
            pl.BlockSpec((blk, D), lambda i: (i, 0)),
        ],
        out_shape=[
            jax.ShapeDtypeStruct((N, 2 * D), jnp.float32),
            jax.ShapeDtypeStruct((N, D), jnp.float32),
        ],
    )(h, wab, b2)


def _edge_body(stbl, btbl, eidx, out,
               sidx, didx, sraw, draw,
               sbuf0, sbuf1, dbuf0, dbuf1, obuf0, obuf1, zbuf, fbuf, wbuf,
               acc, sem_s0, sem_s1, sem_d0, sem_d1, sem_o0, sem_o1):
    cid = lax.axis_index("c")
    sid = lax.axis_index("s")

    pltpu.sync_copy(eidx.at[0, sid], sraw)
    pltpu.sync_copy(eidx.at[1, sid], draw)

    zero16 = jnp.zeros((LANES,), jnp.float32)

    def zrow(r, carry):
        for j in range(2 * CW // LANES):
            zbuf[r, pl.ds(LANES * j, LANES)] = zero16
        return carry

    lax.fori_loop(0, FLUSH, zrow, 0)

    sets = ((sbuf0, dbuf0, obuf0, sem_s0, sem_d0, sem_o0),
            (sbuf1, dbuf1, obuf1, sem_s1, sem_d1, sem_o1))

    def issue(bi, s):
        pltpu.async_copy(stbl.at[sidx.at[bi]], s[0], s[3])
        pltpu.async_copy(btbl.at[didx.at[bi]], s[1], s[4])

    def wait_gathers(bi, s):
        pltpu.make_async_copy(stbl.at[sidx.at[bi]], s[0], s[3]).wait()
        pltpu.make_async_copy(btbl.at[didx.at[bi]], s[1], s[4]).wait()

    def compute(s):
        sb, db, ob = s[0], s[1], s[2]

        def row(r, c2):
            for j in range(CW // LANES):
                sl = pl.ds(LANES * j, LANES)
                slh = pl.ds(CW + LANES * j, LANES)
                x = sb[r, sl] + db[r, sl]
                w = jnp.exp(jnp.maximum(x, x * 0.01))
                ob[r, sl] = w
                ob[r, slh] = sb[r, slh] * w
            return c2

        lax.fori_loop(0, BATCH, row, 0)

    def scatter(bi, s):
        pltpu.async_copy(s[2], acc.at[draw.at[bi]], s[5], add=True)

    def wait_scatter(bi, s):
        pltpu.make_async_copy(s[2], acc.at[draw.at[bi]], s[5]).wait()

    for kc in range(NCHUNK // NC):
        chunk = cid * (NCHUNK // NC) + kc

        def adj(r, carry):
            for j in range(BATCH // LANES):
                sl = pl.ds(LANES * j, LANES)
                sidx[r, sl] = sraw[r, sl] * NCHUNK + chunk
                didx[r, sl] = draw[r, sl] * NCHUNK + chunk
            return carry

        lax.fori_loop(0, NB, adj, 0)

        for z in range(NFLUSH):
            pltpu.sync_copy(zbuf, acc.at[pl.ds(sid * ROWS_PER_SUB + z * FLUSH, FLUSH)])
        plsc.subcore_barrier()

        issue(0, sets[0])
        issue(1, sets[1])

        def pair(bp, carry):
            for ph in range(2):
                s = sets[ph]
                bi = 2 * bp + ph

                @pl.when(bp > 0)
                def _():
                    wait_scatter(bi - 2, s)

                wait_gathers(bi, s)
                compute(s)
                scatter(bi, s)
                issue(jnp.minimum(bi + 2, NB - 1), s)
            return carry

        lax.fori_loop(0, NB // 2, pair, 0)

        wait_scatter(NB - 3, sets[0])
        wait_gathers(NB - 1, sets[0])
        compute(sets[0])
        scatter(NB - 1, sets[0])
        wait_gathers(NB - 1, sets[1])
        wait_scatter(NB - 1, sets[0])
        wait_scatter(NB - 2, sets[1])
        plsc.subcore_barrier()

        for z in range(NFLUSH):
            row0 = sid * ROWS_PER_SUB + z * FLUSH
            pltpu.sync_copy(acc.at[pl.ds(row0, FLUSH)], fbuf)

            def drow(r, carry):
                for j in range(CW // LANES):
                    sl = pl.ds(LANES * j, LANES)
                    den = fbuf[r, sl]
                    num = fbuf[r, pl.ds(CW + LANES * j, LANES)]
                    wbuf[r, sl] = jnp.where(den > 0.0, num / den, 0.0)
                return carry

            lax.fori_loop(0, FLUSH, drow, 0)
            pltpu.sync_copy(wbuf, out.at[chunk, pl.ds(row0, FLUSH)])


_edge_kernel = functools.partial(
    pl.kernel,
    out_type=jax.ShapeDtypeStruct((NCHUNK, NP, CW), jnp.float32),
    mesh=plsc.VectorSubcoreMesh(
        core_axis_name="c", subcore_axis_name="s", num_cores=NC, num_subcores=NS
    ),
    scratch_types=[
        pltpu.VMEM((NB, BATCH), jnp.int32),
        pltpu.VMEM((NB, BATCH), jnp.int32),
        pltpu.VMEM((NB, BATCH), jnp.int32),
        pltpu.VMEM((NB, BATCH), jnp.int32),
        pltpu.VMEM((BATCH, SW), jnp.float32),
        pltpu.VMEM((BATCH, SW), jnp.float32),
        pltpu.VMEM((BATCH, CW), jnp.float32),
        pltpu.VMEM((BATCH, CW), jnp.float32),
        pltpu.VMEM((BATCH, 2 * CW), jnp.float32),
        pltpu.VMEM((BATCH, 2 * CW), jnp.float32),
        pltpu.VMEM((FLUSH, 2 * CW), jnp.float32),
        pltpu.VMEM((FLUSH, 2 * CW), jnp.float32),
        pltpu.VMEM((FLUSH, CW), jnp.float32),
        pltpu.VMEM_SHARED((NP, 2 * CW), jnp.float32),
        pltpu.SemaphoreType.DMA,
        pltpu.SemaphoreType.DMA,
        pltpu.SemaphoreType.DMA,
        pltpu.SemaphoreType.DMA,
        pltpu.SemaphoreType.DMA,
        pltpu.SemaphoreType.DMA,
    ],
    compiler_params=pltpu.CompilerParams(use_tc_tiling_on_sc=False),
)(_edge_body)


def kernel(h, edge_index, W, b):
    wab = jnp.concatenate([W[:, :D].T, W[:, D:].T], axis=1)
    b2 = b.reshape(1, D)
    s, bb = _node_projections(h, wab, b2)
    stbl = s.reshape(NCHUNK * N, SW)
    btbl = bb.reshape(NCHUNK * N, CW)
    eidx = edge_index.reshape(2, NS, NB, BATCH)
    outc = _edge_kernel(stbl, btbl, eidx)
    return outc[:, :N].transpose(1, 0, 2).reshape(N, D)
